# baseline (speedup 1.0000x reference)
import functools

import jax
import jax.numpy as jnp
from jax import lax
from jax.experimental import pallas as pl
from jax.experimental.pallas import tpu as pltpu
from jax.experimental.pallas import tpu_sc as plsc

_B, _F, _V, _D = 16384, 26, 100000, 16
_C = _F * _D
_H1, _H2 = 512, 256
_EPS = 1e-5
_S = 33408
_THIRDS = ((0, _S), (_S, _S), (2 * _S, 33152))
_TAIL = 2 * _S + 33152


def _gather_xt():
    info = plsc.get_sparse_core_info()
    nw = info.num_cores * info.num_subcores
    rows_per_w = _C // nw
    n_stage = 3 * rows_per_w

    mesh = plsc.VectorSubcoreMesh(core_axis_name="c", subcore_axis_name="s")

    @functools.partial(
        pl.kernel,
        mesh=mesh,
        out_type=(
            jax.ShapeDtypeStruct((_C, _B), jnp.float32),
            jax.ShapeDtypeStruct((32 * 1024,), jnp.float32),
        ),
        compiler_params=pltpu.CompilerParams(needs_layout_passes=False),
        scratch_types=[
            pltpu.VMEM((_S,), jnp.float32),
            pltpu.VMEM((_S,), jnp.float32),
            pltpu.VMEM((_B,), jnp.int32),
            pltpu.VMEM((_B,), jnp.float32),
            pltpu.VMEM((_B,), jnp.float32),
            pltpu.VMEM((1024,), jnp.float32),
            pltpu.VMEM((128,), jnp.float32),
            pltpu.SemaphoreType.DMA,
            pltpu.SemaphoreType.DMA,
            pltpu.SemaphoreType.DMA,
            pltpu.SemaphoreType.DMA,
            pltpu.SemaphoreType.DMA,
        ],
    )
    def gather_kernel(idx_hbm, tab_hbm, tail_hbm, out_hbm, stat_hbm,
                      buf_a, buf_b, idx_v, out_a, out_b, sstat_v, tail_v,
                      sem_a, sem_b, sem_o, sem_s, sem_t):
        wid = lax.axis_index("s") * info.num_cores + lax.axis_index("c")
        base = wid * rows_per_w
        bufs = (buf_a, buf_b)
        sems = (sem_a, sem_b)
        outs = (out_a, out_b)
        iota16 = lax.iota(jnp.int32, 16)

        def start_stage(s):
            i, t = divmod(s, 3)
            off, sz = _THIRDS[t]
            return pltpu.async_copy(
                tab_hbm.at[base + i, pl.ds(off, sz)],
                bufs[s % 2].at[pl.ds(0, sz)],
                sems[s % 2],
            )

        pend = [start_stage(0), None]
        out_descs = []

        for i in range(rows_per_w):
            c = base + i
            f = c // _D
            pltpu.sync_copy(idx_hbm.at[f], idx_v)
            tail_desc = pltpu.async_copy(tail_hbm.at[c], tail_v, sem_t)
            out_v = outs[i % 2]
            if i >= 2:
                out_descs[i - 2].wait()

            for t in range(3):
                s = 3 * i + t
                off, sz = _THIRDS[t]
                cur = bufs[s % 2]
                pend[s % 2].wait()
                if s + 1 < n_stage:
                    pend[(s + 1) % 2] = start_stage(s + 1)
                if t == 2:
                    tail_desc.wait()

                @plsc.parallel_loop(0, _B // 16, unroll=8)
                def _(j, cur=cur, out_v=out_v, off=off, sz=sz, t=t):
                    iv = idx_v[pl.ds(j * 16, 16)]
                    ivl = iv - off
                    mask = (ivl >= 0) & (ivl < sz)
                    g = plsc.load_gather(cur, [ivl], mask=mask)
                    plsc.store_scatter(out_v, [j * 16 + iota16], g, mask=mask)
                    if t == 2:
                        ivt = iv - _TAIL
                        maskt = ivt >= 0
                        gt = plsc.load_gather(tail_v, [ivt], mask=maskt)
                        plsc.store_scatter(
                            out_v, [j * 16 + iota16], gt, mask=maskt)

            out_descs.append(
                pltpu.async_copy(out_v, out_hbm.at[c], sem_o))

            zero = jnp.zeros((16,), jnp.float32)

            @plsc.parallel_loop(0, _B // 16, unroll=8, carry=(zero, zero))
            def _(j, cs, out_v=out_v):
                s_, q_ = cs
                v = out_v[pl.ds(j * 16, 16)]
                return (s_ + v, q_ + v * v)

            sum_vec, sq_vec = _
            sstat_v[pl.ds(i * 16, 16)] = sum_vec
            sstat_v[pl.ds(512 + i * 16, 16)] = sq_vec

        pltpu.async_copy(
            sstat_v, stat_hbm.at[pl.ds(wid * 1024, 1024)], sem_s).wait()
        for d in out_descs[-2:]:
            d.wait()

    return gather_kernel


def _mlp(xt, sum16, sq16, gamma, beta, w1t, b1, w2t, b2, w3, b3):
    bb = 2048
    nb = _B // bb
    inv_b = 1.0 / _B

    def body(x_ref, s_ref, q_ref, g_ref, be_ref, w1_ref, b1_ref, w2_ref,
             b2_ref, w3_ref, b3_ref, o_ref):
        mean = jnp.sum(s_ref[...], axis=1, keepdims=True) * inv_b
        var = jnp.sum(q_ref[...], axis=1, keepdims=True) * inv_b - mean * mean
        scale = g_ref[...] * lax.rsqrt(var + _EPS)
        shift = be_ref[...] - mean * scale
        xn_t = x_ref[...] * scale + shift
        h1 = jnp.maximum(
            lax.dot_general(xn_t, w1_ref[...], (((0,), (0,)), ((), ())),
                            preferred_element_type=jnp.float32)
            + b1_ref[...], 0.0)
        h2 = jnp.maximum(
            jnp.dot(h1, w2_ref[...], preferred_element_type=jnp.float32)
            + b2_ref[...], 0.0)
        o = jnp.sum(h2 * w3_ref[...], axis=1, keepdims=True) + b3_ref[...]
        o_ref[...] = 1.0 / (1.0 + jnp.exp(-o))

    full = lambda i: (0, 0)
    return pl.pallas_call(
        body,
        grid=(nb,),
        in_specs=[
            pl.BlockSpec((_C, bb), lambda i: (0, i)),
            pl.BlockSpec((_C, 16), full),
            pl.BlockSpec((_C, 16), full),
            pl.BlockSpec((_C, 1), full),
            pl.BlockSpec((_C, 1), full),
            pl.BlockSpec((_C, _H1), full),
            pl.BlockSpec((1, _H1), full),
            pl.BlockSpec((_H1, _H2), full),
            pl.BlockSpec((1, _H2), full),
            pl.BlockSpec((1, _H2), full),
            pl.BlockSpec((1, 1), full),
        ],
        out_specs=pl.BlockSpec((bb, 1), lambda i: (i, 0)),
        out_shape=jax.ShapeDtypeStruct((_B, 1), jnp.float32),
    )(xt, sum16, sq16, gamma, beta, w1t, b1, w2t, b2, w3, b3)


def kernel(indices, tables, bn_gamma, bn_beta, W1, b1, W2, b2, W3, b3):
    idx_t = jnp.clip(indices, 0, _V - 1).astype(jnp.int32).T
    tab_rows = tables.transpose(0, 2, 1).reshape(_C, _V)
    tail128 = jnp.pad(tab_rows[:, _TAIL:], ((0, 0), (0, 128 - (_V - _TAIL))))
    xt, stat_flat = _gather_xt()(idx_t, tab_rows, tail128)
    st = stat_flat.reshape(32, 2, 32, 16)
    sum16 = st[:, 0, :13, :].reshape(_C, 16)
    sq16 = st[:, 1, :13, :].reshape(_C, 16)
    out = _mlp(
        xt, sum16, sq16,
        bn_gamma.reshape(_C, 1), bn_beta.reshape(_C, 1),
        W1.T, b1.reshape(1, _H1),
        W2.T, b2.reshape(1, _H2),
        W3.reshape(1, _H2), b3.reshape(1, 1),
    )
    return out.reshape(_B)


# speedup vs baseline: 1.1083x; 1.1083x over previous
import functools

import jax
import jax.numpy as jnp
from jax import lax
from jax.experimental import pallas as pl
from jax.experimental.pallas import tpu as pltpu
from jax.experimental.pallas import tpu_sc as plsc

_B, _F, _V, _D = 16384, 26, 100000, 16
_C = _F * _D
_H1, _H2 = 512, 256
_EPS = 1e-5
_H0 = 50048
_H1SZ = 49920
_TAIL = _H0 + _H1SZ
_ICH = 4096


def _gather_xt():
    info = plsc.get_sparse_core_info()
    nw = info.num_cores * info.num_subcores
    rows_per_w = _C // nw
    n_ich = _B // _ICH

    mesh = plsc.VectorSubcoreMesh(core_axis_name="c", subcore_axis_name="s")

    @functools.partial(
        pl.kernel,
        mesh=mesh,
        out_type=(
            jax.ShapeDtypeStruct((_C, _B), jnp.float32),
            jax.ShapeDtypeStruct((32 * 1024,), jnp.float32),
        ),
        compiler_params=pltpu.CompilerParams(needs_layout_passes=False),
        scratch_types=[
            pltpu.VMEM((_H0,), jnp.float32),
            pltpu.VMEM((_H1SZ + 128,), jnp.float32),
            pltpu.VMEM((_ICH,), jnp.int32),
            pltpu.VMEM((_ICH,), jnp.int32),
            pltpu.VMEM((_B,), jnp.float32),
            pltpu.VMEM((1024,), jnp.float32),
            pltpu.SemaphoreType.DMA,
            pltpu.SemaphoreType.DMA,
            pltpu.SemaphoreType.DMA,
            pltpu.SemaphoreType.DMA,
        ],
    )
    def gather_kernel(idx_hbm, tab_hbm, tail_hbm, out_hbm, stat_hbm,
                      buf_a, buf_b, idx_0, idx_1, out_v, sstat_v,
                      sem_a, sem_b, sem_i, sem_o):
        wid = lax.axis_index("s") * info.num_cores + lax.axis_index("c")
        base = wid * rows_per_w
        ibufs = (idx_0, idx_1)
        iota16 = lax.iota(jnp.int32, 16)
        zero = jnp.zeros((16,), jnp.float32)
        out_desc = None

        for i in range(rows_per_w):
            c = base + i
            f = c // _D
            d_a = pltpu.async_copy(
                tab_hbm.at[c, pl.ds(0, _H0)], buf_a, sem_a)
            d_b = pltpu.async_copy(
                tab_hbm.at[c, pl.ds(_H0, _H1SZ)],
                buf_b.at[pl.ds(0, _H1SZ)], sem_b)
            d_t = pltpu.async_copy(
                tail_hbm.at[c], buf_b.at[pl.ds(_H1SZ, 128)], sem_b)
            d_i = pltpu.async_copy(
                idx_hbm.at[f, pl.ds(0, _ICH)], ibufs[0], sem_i)
            if out_desc is not None:
                out_desc.wait()
            d_a.wait()
            d_b.wait()
            d_t.wait()

            carry = (zero, zero)
            for k in range(n_ich):
                d_i.wait()
                if k + 1 < n_ich:
                    d_i = pltpu.async_copy(
                        idx_hbm.at[f, pl.ds((k + 1) * _ICH, _ICH)],
                        ibufs[(k + 1) % 2], sem_i)
                ib = ibufs[k % 2]
                b0 = k * _ICH

                @plsc.parallel_loop(0, _ICH // 16, unroll=8, carry=carry)
                def _(j, cs, ib=ib, b0=b0):
                    s_, q_ = cs
                    iv = ib[pl.ds(j * 16, 16)]
                    m0 = iv < _H0
                    m1 = ~m0
                    g0 = plsc.load_gather(buf_a, [iv], mask=m0)
                    g1 = plsc.load_gather(buf_b, [iv - _H0], mask=m1)
                    g = jnp.where(m0, g0, g1)
                    out_v[pl.ds(b0 + j * 16, 16)] = g
                    return (s_ + g, q_ + g * g)

                carry = _

            out_desc = pltpu.async_copy(out_v, out_hbm.at[c], sem_o)
            sum_vec, sq_vec = carry
            sstat_v[pl.ds(i * 16, 16)] = sum_vec
            sstat_v[pl.ds(512 + i * 16, 16)] = sq_vec

        pltpu.async_copy(
            sstat_v, stat_hbm.at[pl.ds(wid * 1024, 1024)], sem_a).wait()
        out_desc.wait()

    return gather_kernel


def _mlp(xt, sum16, sq16, gamma, beta, w1t, b1, w2t, b2, w3, b3):
    bb = 2048
    nb = _B // bb
    inv_b = 1.0 / _B

    def body(x_ref, s_ref, q_ref, g_ref, be_ref, w1_ref, b1_ref, w2_ref,
             b2_ref, w3_ref, b3_ref, o_ref):
        mean = jnp.sum(s_ref[...], axis=1, keepdims=True) * inv_b
        var = jnp.sum(q_ref[...], axis=1, keepdims=True) * inv_b - mean * mean
        scale = g_ref[...] * lax.rsqrt(var + _EPS)
        shift = be_ref[...] - mean * scale
        xn_t = x_ref[...] * scale + shift
        h1 = jnp.maximum(
            lax.dot_general(xn_t, w1_ref[...], (((0,), (0,)), ((), ())),
                            preferred_element_type=jnp.float32)
            + b1_ref[...], 0.0)
        h2 = jnp.maximum(
            jnp.dot(h1, w2_ref[...], preferred_element_type=jnp.float32)
            + b2_ref[...], 0.0)
        o = jnp.sum(h2 * w3_ref[...], axis=1, keepdims=True) + b3_ref[...]
        o_ref[...] = 1.0 / (1.0 + jnp.exp(-o))

    full = lambda i: (0, 0)
    return pl.pallas_call(
        body,
        grid=(nb,),
        in_specs=[
            pl.BlockSpec((_C, bb), lambda i: (0, i)),
            pl.BlockSpec((_C, 16), full),
            pl.BlockSpec((_C, 16), full),
            pl.BlockSpec((_C, 1), full),
            pl.BlockSpec((_C, 1), full),
            pl.BlockSpec((_C, _H1), full),
            pl.BlockSpec((1, _H1), full),
            pl.BlockSpec((_H1, _H2), full),
            pl.BlockSpec((1, _H2), full),
            pl.BlockSpec((1, _H2), full),
            pl.BlockSpec((1, 1), full),
        ],
        out_specs=pl.BlockSpec((bb, 1), lambda i: (i, 0)),
        out_shape=jax.ShapeDtypeStruct((_B, 1), jnp.float32),
    )(xt, sum16, sq16, gamma, beta, w1t, b1, w2t, b2, w3, b3)


def kernel(indices, tables, bn_gamma, bn_beta, W1, b1, W2, b2, W3, b3):
    idx_t = jnp.clip(indices, 0, _V - 1).astype(jnp.int32).T
    tab_rows = tables.transpose(0, 2, 1).reshape(_C, _V)
    tail128 = jnp.pad(tab_rows[:, _TAIL:], ((0, 0), (0, 128 - (_V - _TAIL))))
    xt, stat_flat = _gather_xt()(idx_t, tab_rows, tail128)
    st = stat_flat.reshape(32, 2, 32, 16)
    sum16 = st[:, 0, :13, :].reshape(_C, 16)
    sq16 = st[:, 1, :13, :].reshape(_C, 16)
    out = _mlp(
        xt, sum16, sq16,
        bn_gamma.reshape(_C, 1), bn_beta.reshape(_C, 1),
        W1.T, b1.reshape(1, _H1),
        W2.T, b2.reshape(1, _H2),
        W3.reshape(1, _H2), b3.reshape(1, 1),
    )
    return out.reshape(_B)


# speedup vs baseline: 1.2547x; 1.1320x over previous
import functools

import jax
import jax.numpy as jnp
from jax import lax
from jax.experimental import pallas as pl
from jax.experimental.pallas import tpu as pltpu
from jax.experimental.pallas import tpu_sc as plsc

_B, _F, _V, _D = 16384, 26, 100000, 16
_C = _F * _D
_H1, _H2 = 512, 256
_EPS = 1e-5
_H0 = 50048
_H1SZ = 49920
_TAIL = _H0 + _H1SZ
_ICH = 4096


def _gather_xt():
    info = plsc.get_sparse_core_info()
    nw = info.num_cores * info.num_subcores
    rows_per_w = _C // nw
    n_ich = _B // _ICH

    mesh = plsc.VectorSubcoreMesh(core_axis_name="c", subcore_axis_name="s")

    @functools.partial(
        pl.kernel,
        mesh=mesh,
        out_type=(
            jax.ShapeDtypeStruct((_C, _B), jnp.float32),
            jax.ShapeDtypeStruct((32 * 1024,), jnp.float32),
        ),
        compiler_params=pltpu.CompilerParams(needs_layout_passes=False),
        scratch_types=[
            pltpu.VMEM((_H0,), jnp.float32),
            pltpu.VMEM((_H1SZ + 128,), jnp.float32),
            pltpu.VMEM((_B,), jnp.int32),
            pltpu.VMEM((_ICH,), jnp.float32),
            pltpu.VMEM((_ICH,), jnp.float32),
            pltpu.VMEM((1024,), jnp.float32),
            pltpu.SemaphoreType.DMA,
            pltpu.SemaphoreType.DMA,
            pltpu.SemaphoreType.DMA,
        ],
    )
    def gather_kernel(idx_hbm, tab_hbm, tail_hbm, out_hbm, stat_hbm,
                      buf_a, buf_b, idx_v, out_0, out_1, sstat_v,
                      sem_a, sem_b, sem_o):
        wid = lax.axis_index("s") * info.num_cores + lax.axis_index("c")
        base = wid * rows_per_w
        obufs = (out_0, out_1)
        zero = jnp.zeros((16,), jnp.float32)
        out_descs = [None, None]

        for i in range(rows_per_w):
            c = base + i
            f = c // _D
            d_a = pltpu.async_copy(
                tab_hbm.at[c, pl.ds(0, _H0)], buf_a, sem_a)
            d_b = pltpu.async_copy(
                tab_hbm.at[c, pl.ds(_H0, _H1SZ)],
                buf_b.at[pl.ds(0, _H1SZ)], sem_b)
            d_t = pltpu.async_copy(
                tail_hbm.at[c], buf_b.at[pl.ds(_H1SZ, 128)], sem_b)
            if i == 0:
                pltpu.sync_copy(idx_hbm.at[f], idx_v)
            else:
                @pl.when(f != (c - 1) // _D)
                def _():
                    pltpu.sync_copy(idx_hbm.at[f], idx_v)
            d_a.wait()
            d_b.wait()
            d_t.wait()

            carry = (zero, zero)
            for k in range(n_ich):
                ob = obufs[k % 2]
                b0 = k * _ICH
                if out_descs[k % 2] is not None:
                    out_descs[k % 2].wait()

                @plsc.parallel_loop(0, _ICH // 16, unroll=8, carry=carry)
                def _(j, cs, ob=ob, b0=b0):
                    s_, q_ = cs
                    iv = idx_v[pl.ds(b0 + j * 16, 16)]
                    m0 = iv < _H0
                    m1 = ~m0
                    g0 = plsc.load_gather(buf_a, [iv], mask=m0)
                    g1 = plsc.load_gather(buf_b, [iv - _H0], mask=m1)
                    g = jnp.where(m0, g0, g1)
                    ob[pl.ds(j * 16, 16)] = g
                    return (s_ + g, q_ + g * g)

                carry = _
                out_descs[k % 2] = pltpu.async_copy(
                    ob, out_hbm.at[c, pl.ds(b0, _ICH)], sem_o)

            sum_vec, sq_vec = carry
            sstat_v[pl.ds(i * 16, 16)] = sum_vec
            sstat_v[pl.ds(512 + i * 16, 16)] = sq_vec

        pltpu.async_copy(
            sstat_v, stat_hbm.at[pl.ds(wid * 1024, 1024)], sem_a).wait()
        out_descs[0].wait()
        out_descs[1].wait()

    return gather_kernel


def _mlp(xt, sum16, sq16, gamma, beta, w1t, b1, w2t, b2, w3, b3):
    bb = 2048
    nb = _B // bb
    inv_b = 1.0 / _B

    def body(x_ref, s_ref, q_ref, g_ref, be_ref, w1_ref, b1_ref, w2_ref,
             b2_ref, w3_ref, b3_ref, o_ref):
        mean = jnp.sum(s_ref[...], axis=1, keepdims=True) * inv_b
        var = jnp.sum(q_ref[...], axis=1, keepdims=True) * inv_b - mean * mean
        scale = g_ref[...] * lax.rsqrt(var + _EPS)
        shift = be_ref[...] - mean * scale
        xn_t = x_ref[...] * scale + shift
        h1 = jnp.maximum(
            lax.dot_general(xn_t, w1_ref[...], (((0,), (0,)), ((), ())),
                            preferred_element_type=jnp.float32)
            + b1_ref[...], 0.0)
        h2 = jnp.maximum(
            jnp.dot(h1, w2_ref[...], preferred_element_type=jnp.float32)
            + b2_ref[...], 0.0)
        o = jnp.sum(h2 * w3_ref[...], axis=1, keepdims=True) + b3_ref[...]
        o_ref[...] = 1.0 / (1.0 + jnp.exp(-o))

    full = lambda i: (0, 0)
    return pl.pallas_call(
        body,
        grid=(nb,),
        in_specs=[
            pl.BlockSpec((_C, bb), lambda i: (0, i)),
            pl.BlockSpec((_C, 16), full),
            pl.BlockSpec((_C, 16), full),
            pl.BlockSpec((_C, 1), full),
            pl.BlockSpec((_C, 1), full),
            pl.BlockSpec((_C, _H1), full),
            pl.BlockSpec((1, _H1), full),
            pl.BlockSpec((_H1, _H2), full),
            pl.BlockSpec((1, _H2), full),
            pl.BlockSpec((1, _H2), full),
            pl.BlockSpec((1, 1), full),
        ],
        out_specs=pl.BlockSpec((bb, 1), lambda i: (i, 0)),
        out_shape=jax.ShapeDtypeStruct((_B, 1), jnp.float32),
    )(xt, sum16, sq16, gamma, beta, w1t, b1, w2t, b2, w3, b3)


def kernel(indices, tables, bn_gamma, bn_beta, W1, b1, W2, b2, W3, b3):
    idx_t = jnp.clip(indices, 0, _V - 1).astype(jnp.int32).T
    tab_rows = tables.transpose(0, 2, 1).reshape(_C, _V)
    tail128 = jnp.pad(tab_rows[:, _TAIL:], ((0, 0), (0, 128 - (_V - _TAIL))))
    xt, stat_flat = _gather_xt()(idx_t, tab_rows, tail128)
    st = stat_flat.reshape(32, 2, 32, 16)
    sum16 = st[:, 0, :13, :].reshape(_C, 16)
    sq16 = st[:, 1, :13, :].reshape(_C, 16)
    out = _mlp(
        xt, sum16, sq16,
        bn_gamma.reshape(_C, 1), bn_beta.reshape(_C, 1),
        W1.T, b1.reshape(1, _H1),
        W2.T, b2.reshape(1, _H2),
        W3.reshape(1, _H2), b3.reshape(1, 1),
    )
    return out.reshape(_B)
